# baseline (speedup 1.0000x reference)
import functools

import jax
import jax.numpy as jnp
from jax import lax
from jax.experimental import pallas as pl
from jax.experimental.pallas import tpu as pltpu
from jax.experimental.pallas import tpu_sc as plsc

NC, NS, LANES = 2, 16, 16
NW = NC * NS
_SC_PARAMS = pltpu.CompilerParams(needs_layout_passes=False)


def _pre_body(h_ref, wa_ref, wb_ref, a_ref, b_ref):
    hb = h_ref[...]
    a_ref[...] = jnp.dot(hb, wa_ref[...], preferred_element_type=jnp.float32)
    b_ref[...] = jnp.dot(hb, wb_ref[...], preferred_element_type=jnp.float32)


def _pre_tc(h, wa, wb, bn=400):
    N, D = h.shape
    H = wa.shape[1]
    return pl.pallas_call(
        _pre_body,
        grid=(N // bn,),
        in_specs=[
            pl.BlockSpec((bn, D), lambda i: (i, 0)),
            pl.BlockSpec((D, H), lambda i: (0, 0)),
            pl.BlockSpec((D, H), lambda i: (0, 0)),
        ],
        out_specs=[
            pl.BlockSpec((bn, H), lambda i: (i, 0)),
            pl.BlockSpec((bn, H), lambda i: (i, 0)),
        ],
        out_shape=[
            jax.ShapeDtypeStruct((N, H), jnp.float32),
            jax.ShapeDtypeStruct((N, H), jnp.float32),
        ],
    )(h, wa, wb)


def _gather_sc(A, B, xa, xb, xc, rowp, colp, C=128):
    N, H = A.shape
    Ep = rowp.shape[0]
    EW = Ep // NW
    NCH = EW // C
    mesh = plsc.VectorSubcoreMesh(core_axis_name="c", subcore_axis_name="s")

    @functools.partial(
        pl.kernel,
        out_type=[
            jax.ShapeDtypeStruct((Ep, H), jnp.float32),
            jax.ShapeDtypeStruct((Ep, H), jnp.float32),
            jax.ShapeDtypeStruct((Ep, LANES), jnp.float32),
        ],
        mesh=mesh,
        compiler_params=_SC_PARAMS,
        scratch_types=[
            pltpu.VMEM((EW,), jnp.int32),
            pltpu.VMEM((EW,), jnp.int32),
            pltpu.VMEM((N,), jnp.float32),
            pltpu.VMEM((N,), jnp.float32),
            pltpu.VMEM((N,), jnp.float32),
            pltpu.VMEM((C, LANES), jnp.float32),
            pltpu.VMEM((C, H), jnp.float32),
            pltpu.VMEM((C, H), jnp.float32),
            pltpu.VMEM((C, H), jnp.float32),
            pltpu.VMEM((C, H), jnp.float32),
        ] + [pltpu.SemaphoreType.DMA] * 8,
    )
    def k(a_hbm, b_hbm, xa_hbm, xb_hbm, xc_hbm, row_hbm, col_hbm,
          ga_hbm, gb_hbm, geo_hbm,
          idxr_v, idxc_v, xa_v, xb_v, xc_v, geo_v,
          buf_a0, buf_a1, buf_b0, buf_b1,
          gsa0, gsa1, gsb0, gsb1, wsa0, wsa1, wsb0, wsb1):
        wid = lax.axis_index("s") * NC + lax.axis_index("c")
        base = wid * EW
        pltpu.sync_copy(row_hbm.at[pl.ds(base, EW)], idxr_v)
        pltpu.sync_copy(col_hbm.at[pl.ds(base, EW)], idxc_v)
        pltpu.sync_copy(xa_hbm, xa_v)
        pltpu.sync_copy(xb_hbm, xb_v)
        pltpu.sync_copy(xc_hbm, xc_v)
        lane = lax.iota(jnp.int32, LANES)
        xs = [xa_v, xb_v, xc_v]

        def geo_chunk(c):
            def grp(gi, _):
                e0 = c * C + gi * LANES
                ir = idxr_v[pl.ds(e0, LANES)]
                ic = idxc_v[pl.ds(e0, LANES)]
                g = gi * LANES + lane
                rad = jnp.zeros((LANES,), jnp.float32)
                for d in range(3):
                    dd = jnp.full((LANES,), d, jnp.int32)
                    diff = (plsc.load_gather(xs[d], [ir])
                            - plsc.load_gather(xs[d], [ic]))
                    plsc.store_scatter(geo_v, [g, dd], diff)
                    rad = rad + diff * diff
                plsc.store_scatter(
                    geo_v, [g, jnp.full((LANES,), 3, jnp.int32)], rad)
                return 0
            lax.fori_loop(0, C // LANES, grp, 0)
            pltpu.sync_copy(geo_v, geo_hbm.at[pl.ds(base + c * C, C)])
        bufs_a = [buf_a0, buf_a1]
        bufs_b = [buf_b0, buf_b1]
        gsa = [gsa0, gsa1]
        gsb = [gsb0, gsb1]
        wsa = [wsa0, wsa1]
        wsb = [wsb0, wsb1]

        def g_cp(c, b):
            off = c * C
            return (pltpu.make_async_copy(
                        a_hbm.at[idxr_v.at[pl.ds(off, C)]], bufs_a[b], gsa[b]),
                    pltpu.make_async_copy(
                        b_hbm.at[idxc_v.at[pl.ds(off, C)]], bufs_b[b], gsb[b]))

        def w_cp(c, b):
            off = base + c * C
            return (pltpu.make_async_copy(
                        bufs_a[b], ga_hbm.at[pl.ds(off, C)], wsa[b]),
                    pltpu.make_async_copy(
                        bufs_b[b], gb_hbm.at[pl.ds(off, C)], wsb[b]))

        for cp in g_cp(0, 0):
            cp.start()

        def outer(s, _):
            for b in (0, 1):
                c = 2 * s + b
                for cp in g_cp(c, b):
                    cp.wait()
                for cp in w_cp(c, b):
                    cp.start()

                @pl.when(c + 1 < NCH)
                def _():
                    @pl.when(c >= 1)
                    def _():
                        for cp in w_cp(c - 1, 1 - b):
                            cp.wait()
                    for cp in g_cp(c + 1, 1 - b):
                        cp.start()

                geo_chunk(c)
            return 0

        lax.fori_loop(0, NCH // 2, outer, 0)
        for cp in w_cp(NCH - 2, 0):
            cp.wait()
        for cp in w_cp(NCH - 1, 1):
            cp.wait()

    return k(A, B, xa, xb, xc, rowp, colp)


def _edge_body(nedge, ga_ref, gb_ref, geo_ref, ea_ref,
               we1e_ref, we1r_ref, be1_ref, we2_ref, be2_ref,
               wc1_ref, bc1_ref, wc2_ref,
               ef_ref, t16_ref):
    be = ga_ref.shape[0]
    geo = geo_ref[...]
    radial = geo[:, 3:4]
    m = jnp.maximum(
        ga_ref[...] + gb_ref[...] + radial * we1r_ref[...]
        + jnp.dot(ea_ref[...], we1e_ref[...],
                  preferred_element_type=jnp.float32)
        + be1_ref[...], 0.0)
    ef = jnp.maximum(
        jnp.dot(m.astype(jnp.bfloat16), we2_ref[...],
                preferred_element_type=jnp.float32)
        + be2_ref[...], 0.0)
    ch = jnp.maximum(
        jnp.dot(ef.astype(jnp.bfloat16), wc1_ref[...],
                preferred_element_type=jnp.float32)
        + bc1_ref[...], 0.0)
    cm = jnp.dot(ch.astype(jnp.bfloat16), wc2_ref[...],
                 preferred_element_type=jnp.float32)
    t = jnp.clip(geo * cm, -100.0, 100.0)
    lane = lax.broadcasted_iota(jnp.int32, t.shape, 1)
    t16 = jnp.where(lane < 3, t, jnp.where(lane == 3, 1.0, 0.0))
    eid = pl.program_id(0) * be + lax.broadcasted_iota(jnp.int32, (be, 1), 0)
    emask = eid < nedge
    ef_ref[...] = jnp.where(emask, ef, 0.0)
    t16_ref[...] = jnp.where(emask, t16, 0.0)


def _edge_tc(nedge, ga, gb, geo, ea,
             we1e, we1r, be1, we2, be2, wc1, bc1, wc2, be=512):
    Ep, H = ga.shape
    DE = ea.shape[1]
    ea_last = (ea.shape[0] + be - 1) // be - 1
    full = lambda shape: pl.BlockSpec(shape, lambda i: (0, 0))
    return pl.pallas_call(
        functools.partial(_edge_body, nedge),
        grid=(Ep // be,),
        in_specs=[
            pl.BlockSpec((be, H), lambda i: (i, 0)),
            pl.BlockSpec((be, H), lambda i: (i, 0)),
            pl.BlockSpec((be, LANES), lambda i: (i, 0)),
            pl.BlockSpec((be, DE), lambda i: (jnp.minimum(i, ea_last), 0)),
            full((DE, H)), full((1, H)), full((1, H)),
            full((H, H)), full((1, H)),
            full((H, H)), full((1, H)), full((H, 1)),
        ],
        out_specs=[
            pl.BlockSpec((be, H), lambda i: (i, 0)),
            pl.BlockSpec((be, LANES), lambda i: (i, 0)),
        ],
        out_shape=[
            jax.ShapeDtypeStruct((Ep, H), jnp.float32),
            jax.ShapeDtypeStruct((Ep, LANES), jnp.float32),
        ],
    )(ga, gb, geo, ea,
      we1e, we1r, be1, we2, be2, wc1, bc1, wc2)


def _scatter_sc(ef, t16, row2d, col2d, z128, z1d, C=128):
    H = ef.shape[1]
    N = 10 * z128.shape[0]
    Ep = row2d.shape[0] * C
    ET = Ep // NS
    NCH = ET // C
    ZR = z128.shape[0]
    mesh = plsc.VectorSubcoreMesh(core_axis_name="c", subcore_axis_name="s")

    @functools.partial(
        pl.kernel,
        out_type=[
            jax.ShapeDtypeStruct((N, H), jnp.float32),
            jax.ShapeDtypeStruct((N, H), jnp.float32),
            jax.ShapeDtypeStruct((4, N), jnp.float32),
        ],
        mesh=mesh,
        compiler_params=_SC_PARAMS,
        scratch_types=[
            pltpu.VMEM((NCH, C), jnp.int32),
            pltpu.VMEM((C, H), jnp.float32),
            pltpu.VMEM((C, H), jnp.float32),
            pltpu.VMEM((C, LANES), jnp.float32),
            pltpu.VMEM((C,), jnp.float32),
            pltpu.VMEM((C,), jnp.float32),
            pltpu.VMEM((C,), jnp.float32),
            pltpu.VMEM((C,), jnp.float32),
            pltpu.VMEM_SHARED((N, H), jnp.float32),
            pltpu.VMEM_SHARED((N,), jnp.float32),
            pltpu.VMEM_SHARED((N,), jnp.float32),
            pltpu.VMEM_SHARED((N,), jnp.float32),
            pltpu.VMEM_SHARED((N,), jnp.float32),
        ] + [pltpu.SemaphoreType.DMA] * 4,
    )
    def k(ef_hbm, t16_hbm, row2d_hbm, col2d_hbm, z128_hbm, z1d_hbm,
          agg_hbm, oth_hbm, fcnt_hbm,
          idx2_v, fbuf0, fbuf1, tbuf, tc0, tc1, tc2, tc3,
          shf, sh0, sh1, sh2, sh3,
          ls0, ls1, lt0, lt1):
        cid = lax.axis_index("c")
        sid = lax.axis_index("s")
        shcs = [sh0, sh1, sh2, sh3]
        tcs = [tc0, tc1, tc2, tc3]
        zrows = pl.ds(sid * ZR, ZR)
        fbufs = [fbuf0, fbuf1]
        lss = [ls0, ls1]
        lane = lax.iota(jnp.int32, LANES)

        @pl.when(sid < 10)
        def _():
            pltpu.sync_copy(z128_hbm, shf.at[zrows])

        @pl.when(cid == 0)
        def _():
            for d in range(4):
                @pl.when(sid == d)
                def _():
                    pltpu.sync_copy(z1d_hbm, shcs[d])

        plsc.subcore_barrier()

        def run(idx_hbm, do_t):
            pltpu.sync_copy(idx_hbm.at[pl.ds(sid * NCH, NCH)], idx2_v)

            def f_cp(c, b):
                off = sid * ET + c * C
                return pltpu.make_async_copy(
                    ef_hbm.at[pl.ds(off, C)], fbufs[b], lss[b])

            def t_cp(c):
                off = sid * ET + c * C
                return pltpu.make_async_copy(
                    t16_hbm.at[pl.ds(off, C)], tbuf, lt0)

            f_cp(0, 0).start()
            if do_t:
                t_cp(0).start()

            def outer(s, _):
                for b in (0, 1):
                    c = 2 * s + b
                    f_cp(c, b).wait()

                    @pl.when(c + 1 < NCH)
                    def _():
                        f_cp(c + 1, 1 - b).start()

                    if do_t:
                        t_cp(c).wait()
                        def comp(j, _):
                            g = j * LANES + lane
                            for d in range(4):
                                dd = jnp.full((LANES,), d, jnp.int32)
                                tcs[d][pl.ds(j * LANES, LANES)] = (
                                    plsc.load_gather(tbuf, [g, dd]))
                            return 0
                        lax.fori_loop(0, C // LANES, comp, 0)

                        @pl.when(c + 1 < NCH)
                        def _():
                            t_cp(c + 1).start()
                    pltpu.sync_copy(fbufs[b], shf.at[idx2_v.at[c]],
                                    add=True)
                    if do_t:
                        for d in range(4):
                            pltpu.sync_copy(tcs[d],
                                            shcs[d].at[idx2_v.at[c]],
                                            add=True)
                return 0

            lax.fori_loop(0, NCH // 2, outer, 0)

        @pl.when(cid == 0)
        def _():
            run(row2d_hbm, True)

        @pl.when(cid == 1)
        def _():
            run(col2d_hbm, False)

        plsc.subcore_barrier()

        @pl.when(sid < 10)
        def _():
            @pl.when(cid == 0)
            def _():
                pltpu.sync_copy(shf.at[zrows], agg_hbm.at[zrows])

            @pl.when(cid == 1)
            def _():
                pltpu.sync_copy(shf.at[zrows], oth_hbm.at[zrows])

        @pl.when(jnp.logical_and(cid == 0, sid < 4))
        def _():
            for d in range(4):
                @pl.when(sid == d)
                def _():
                    pltpu.sync_copy(shcs[d], fcnt_hbm.at[d])

    return k(ef, t16, row2d, col2d, z128, z1d)


def _node_body(h_ref, x_ref, v_ref, agg_ref, oth_ref, fc_ref,
               wv1_ref, bv1_ref, wv2_ref, bv2_ref,
               wn1a_ref, wn1b_ref, wn1c_ref, bn1_ref, wn2_ref, bn2_ref,
               h_out, x_out, v_out):
    h = h_ref[...]
    fc = fc_ref[...]
    deg = jnp.maximum(fc[:, 3:4], 1.0)
    f = fc[:, 0:3] / deg
    sh = jnp.maximum(
        jnp.dot(h, wv1_ref[...], preferred_element_type=jnp.float32)
        + bv1_ref[...], 0.0)
    scale = jnp.dot(sh, wv2_ref[...],
                    preferred_element_type=jnp.float32) + bv2_ref[...]
    vn = scale * v_ref[...] + f
    v_out[...] = vn
    x_out[...] = x_ref[...] + vn
    nm = jnp.maximum(
        jnp.dot(oth_ref[...], wn1a_ref[...],
                preferred_element_type=jnp.float32)
        + jnp.dot(h, wn1b_ref[...], preferred_element_type=jnp.float32)
        + jnp.dot(agg_ref[...], wn1c_ref[...],
                  preferred_element_type=jnp.float32)
        + bn1_ref[...], 0.0)
    h_out[...] = h + jnp.dot(nm, wn2_ref[...],
                             preferred_element_type=jnp.float32) + bn2_ref[...]


def _node_tc(h, x, v, agg, oth, fcnt,
             wv1, bv1, wv2, bv2, wn1a, wn1b, wn1c, bn1, wn2, bn2, bn=400):
    N, D = h.shape
    H = wn2.shape[0]
    full = lambda shape: pl.BlockSpec(shape, lambda i: (0, 0))
    return pl.pallas_call(
        _node_body,
        grid=(N // bn,),
        in_specs=[
            pl.BlockSpec((bn, D), lambda i: (i, 0)),
            pl.BlockSpec((bn, 3), lambda i: (i, 0)),
            pl.BlockSpec((bn, 3), lambda i: (i, 0)),
            pl.BlockSpec((bn, H), lambda i: (i, 0)),
            pl.BlockSpec((bn, H), lambda i: (i, 0)),
            pl.BlockSpec((bn, 4), lambda i: (i, 0)),
            full((D, H)), full((1, H)), full((H, 1)), full((1, 1)),
            full((H, H)), full((D, H)), full((H, H)), full((1, H)),
            full((H, D)), full((1, D)),
        ],
        out_specs=[
            pl.BlockSpec((bn, D), lambda i: (i, 0)),
            pl.BlockSpec((bn, 3), lambda i: (i, 0)),
            pl.BlockSpec((bn, 3), lambda i: (i, 0)),
        ],
        out_shape=[
            jax.ShapeDtypeStruct((N, D), jnp.float32),
            jax.ShapeDtypeStruct((N, 3), jnp.float32),
            jax.ShapeDtypeStruct((N, 3), jnp.float32),
        ],
    )(h, x, v, agg, oth, fcnt,
      wv1, bv1, wv2, bv2, wn1a, wn1b, wn1c, bn1, wn2, bn2)


def kernel(h, x, v, edge_attr, We1, be1, We2, be2, Wc1, bc1, Wc2,
           Wv1, bv1, Wv2, bv2, Wn1, bn1, Wn2, bn2,
           edge_index, isolated_index):
    N, D = h.shape
    H = We2.shape[0]
    E = edge_index.shape[1]
    DE = edge_attr.shape[1]
    row, col = edge_index[0], edge_index[1]

    C = 128
    Ep = -(-E // (NW * C)) * (NW * C)
    pad = Ep - E
    rowp = jnp.concatenate([row, jnp.zeros((pad,), jnp.int32)])
    colp = jnp.concatenate([col, jnp.zeros((pad,), jnp.int32)])
    xa, xb, xc = x[:, 0], x[:, 1], x[:, 2]
    bf = jnp.bfloat16

    A, Bm = _pre_tc(h, We1[:D], We1[D:2 * D])
    ga, gb, geo = _gather_sc(A, Bm, xa, xb, xc, rowp, colp, C=C)
    ef, t16 = _edge_tc(E, ga, gb, geo, edge_attr,
                       We1[2 * D + 1:], We1[2 * D:2 * D + 1],
                       be1.reshape(1, H), We2.astype(bf),
                       be2.reshape(1, H),
                       Wc1.astype(bf), bc1.reshape(1, H), Wc2.astype(bf))
    z128 = jnp.zeros((N // 10, H), jnp.float32)
    z1d = jnp.zeros((N,), jnp.float32)
    CS = 64
    agg, oth, fcnt = _scatter_sc(ef, t16, rowp.reshape(-1, CS),
                                 colp.reshape(-1, CS), z128, z1d, C=CS)
    return _node_tc(h, x, v, agg, oth, fcnt.T,
                    Wv1, bv1.reshape(1, H), Wv2, bv2.reshape(1, 1),
                    Wn1[:H], Wn1[H:H + D], Wn1[H + D:],
                    bn1.reshape(1, H), Wn2, bn2.reshape(1, D))


# speedup vs baseline: 1.0593x; 1.0593x over previous
import functools

import jax
import jax.numpy as jnp
from jax import lax
from jax.experimental import pallas as pl
from jax.experimental.pallas import tpu as pltpu
from jax.experimental.pallas import tpu_sc as plsc

NC, NS, LANES = 2, 16, 16
NW = NC * NS
_SC_PARAMS = pltpu.CompilerParams(needs_layout_passes=False)


def _pre_body(h_ref, wa_ref, wb_ref, a_ref, b_ref):
    hb = h_ref[...]
    a_ref[...] = jnp.dot(hb, wa_ref[...], preferred_element_type=jnp.float32)
    b_ref[...] = jnp.dot(hb, wb_ref[...], preferred_element_type=jnp.float32)


def _pre_tc(h, wa, wb, bn=400):
    N, D = h.shape
    H = wa.shape[1]
    return pl.pallas_call(
        _pre_body,
        grid=(N // bn,),
        in_specs=[
            pl.BlockSpec((bn, D), lambda i: (i, 0)),
            pl.BlockSpec((D, H), lambda i: (0, 0)),
            pl.BlockSpec((D, H), lambda i: (0, 0)),
        ],
        out_specs=[
            pl.BlockSpec((bn, H), lambda i: (i, 0)),
            pl.BlockSpec((bn, H), lambda i: (i, 0)),
        ],
        out_shape=[
            jax.ShapeDtypeStruct((N, H), jnp.float32),
            jax.ShapeDtypeStruct((N, H), jnp.float32),
        ],
    )(h, wa, wb)


def _gather_sc(A, B, xa, xb, xc, rowp, colp, C=128, nch0=54):
    N, H = A.shape
    Ep = rowp.shape[0]
    NCHT = Ep // (NS * C)
    NCH0 = nch0
    NCH1 = NCHT - NCH0
    EW0, EW1 = NCH0 * C, NCH1 * C
    EWMAX = max(EW0, EW1)
    mesh = plsc.VectorSubcoreMesh(core_axis_name="c", subcore_axis_name="s")

    @functools.partial(
        pl.kernel,
        out_type=[
            jax.ShapeDtypeStruct((Ep, H), jnp.float32),
            jax.ShapeDtypeStruct((Ep, H), jnp.float32),
            jax.ShapeDtypeStruct((Ep, LANES), jnp.float32),
        ],
        mesh=mesh,
        compiler_params=_SC_PARAMS,
        scratch_types=[
            pltpu.VMEM((EWMAX,), jnp.int32),
            pltpu.VMEM((EWMAX,), jnp.int32),
            pltpu.VMEM((N,), jnp.float32),
            pltpu.VMEM((N,), jnp.float32),
            pltpu.VMEM((N,), jnp.float32),
            pltpu.VMEM((C, LANES), jnp.float32),
            pltpu.VMEM((C, H), jnp.float32),
            pltpu.VMEM((C, H), jnp.float32),
            pltpu.VMEM((C, H), jnp.float32),
            pltpu.VMEM((C, H), jnp.float32),
        ] + [pltpu.SemaphoreType.DMA] * 8,
    )
    def k(a_hbm, b_hbm, xa_hbm, xb_hbm, xc_hbm, row_hbm, col_hbm,
          ga_hbm, gb_hbm, geo_hbm,
          idxr_v, idxc_v, xa_v, xb_v, xc_v, geo_v,
          buf_a0, buf_a1, buf_b0, buf_b1,
          gsa0, gsa1, gsb0, gsb1, wsa0, wsa1, wsb0, wsb1):
        cid = lax.axis_index("c")
        sid = lax.axis_index("s")
        pltpu.sync_copy(xa_hbm, xa_v)
        pltpu.sync_copy(xb_hbm, xb_v)
        pltpu.sync_copy(xc_hbm, xc_v)
        lane = lax.iota(jnp.int32, LANES)
        xs = [xa_v, xb_v, xc_v]
        bufs_a = [buf_a0, buf_a1]
        bufs_b = [buf_b0, buf_b1]
        gsa = [gsa0, gsa1]
        gsb = [gsb0, gsb1]
        wsa = [wsa0, wsa1]
        wsb = [wsb0, wsb1]

        def pipeline(base, ew, nch):
            pltpu.sync_copy(row_hbm.at[pl.ds(base, ew)], idxr_v.at[pl.ds(0, ew)])
            pltpu.sync_copy(col_hbm.at[pl.ds(base, ew)], idxc_v.at[pl.ds(0, ew)])

            def geo_chunk(c):
                def grp(gi, _):
                    e0 = c * C + gi * LANES
                    ir = idxr_v[pl.ds(e0, LANES)]
                    ic = idxc_v[pl.ds(e0, LANES)]
                    g = gi * LANES + lane
                    rad = jnp.zeros((LANES,), jnp.float32)
                    for d in range(3):
                        dd = jnp.full((LANES,), d, jnp.int32)
                        diff = (plsc.load_gather(xs[d], [ir])
                                - plsc.load_gather(xs[d], [ic]))
                        plsc.store_scatter(geo_v, [g, dd], diff)
                        rad = rad + diff * diff
                    plsc.store_scatter(
                        geo_v, [g, jnp.full((LANES,), 3, jnp.int32)], rad)
                    return 0
                lax.fori_loop(0, C // LANES, grp, 0)
                pltpu.sync_copy(geo_v, geo_hbm.at[pl.ds(base + c * C, C)])

            def g_cp(c, b):
                off = c * C
                return (pltpu.make_async_copy(
                            a_hbm.at[idxr_v.at[pl.ds(off, C)]],
                            bufs_a[b], gsa[b]),
                        pltpu.make_async_copy(
                            b_hbm.at[idxc_v.at[pl.ds(off, C)]],
                            bufs_b[b], gsb[b]))

            def w_cp(c, b):
                off = base + c * C
                return (pltpu.make_async_copy(
                            bufs_a[b], ga_hbm.at[pl.ds(off, C)], wsa[b]),
                        pltpu.make_async_copy(
                            bufs_b[b], gb_hbm.at[pl.ds(off, C)], wsb[b]))

            for cp in g_cp(0, 0):
                cp.start()

            def outer(s, _):
                for b in (0, 1):
                    c = 2 * s + b
                    for cp in g_cp(c, b):
                        cp.wait()
                    for cp in w_cp(c, b):
                        cp.start()

                    @pl.when(c + 1 < nch)
                    def _():
                        @pl.when(c >= 1)
                        def _():
                            for cp in w_cp(c - 1, 1 - b):
                                cp.wait()
                        for cp in g_cp(c + 1, 1 - b):
                            cp.start()

                    geo_chunk(c)
                return 0

            lax.fori_loop(0, nch // 2, outer, 0)
            for cp in w_cp(nch - 2, 0):
                cp.wait()
            for cp in w_cp(nch - 1, 1):
                cp.wait()

        @pl.when(cid == 0)
        def _():
            pipeline(sid * (EW0 + EW1), EW0, NCH0)

        @pl.when(cid == 1)
        def _():
            pipeline(sid * (EW0 + EW1) + EW0, EW1, NCH1)

    return k(A, B, xa, xb, xc, rowp, colp)


def _edge_body(nedge, ga_ref, gb_ref, geo_ref, ea_ref,
               we1e_ref, we1r_ref, be1_ref, we2_ref, be2_ref,
               wc1_ref, bc1_ref, wc2_ref,
               ef_ref, t16_ref):
    be = ga_ref.shape[0]
    geo = geo_ref[...]
    radial = geo[:, 3:4]
    m = jnp.maximum(
        ga_ref[...] + gb_ref[...] + radial * we1r_ref[...]
        + jnp.dot(ea_ref[...], we1e_ref[...],
                  preferred_element_type=jnp.float32)
        + be1_ref[...], 0.0)
    ef = jnp.maximum(
        jnp.dot(m.astype(jnp.bfloat16), we2_ref[...],
                preferred_element_type=jnp.float32)
        + be2_ref[...], 0.0)
    ch = jnp.maximum(
        jnp.dot(ef.astype(jnp.bfloat16), wc1_ref[...],
                preferred_element_type=jnp.float32)
        + bc1_ref[...], 0.0)
    cm = jnp.dot(ch.astype(jnp.bfloat16), wc2_ref[...],
                 preferred_element_type=jnp.float32)
    t = jnp.clip(geo[:, 0:4] * cm, -100.0, 100.0)
    lane = lax.broadcasted_iota(jnp.int32, t.shape, 1)
    t4 = jnp.where(lane < 3, t, 1.0)
    eid = pl.program_id(0) * be + lax.broadcasted_iota(jnp.int32, (be, 1), 0)
    emask = eid < nedge
    ef_ref[...] = jnp.where(emask, ef, 0.0)
    t16_ref[...] = jnp.where(emask, t4, 0.0)


def _edge_tc(nedge, ga, gb, geo, ea,
             we1e, we1r, be1, we2, be2, wc1, bc1, wc2, be=512):
    Ep, H = ga.shape
    DE = ea.shape[1]
    ea_last = (ea.shape[0] + be - 1) // be - 1
    full = lambda shape: pl.BlockSpec(shape, lambda i: (0, 0))
    return pl.pallas_call(
        functools.partial(_edge_body, nedge),
        grid=(Ep // be,),
        in_specs=[
            pl.BlockSpec((be, H), lambda i: (i, 0)),
            pl.BlockSpec((be, H), lambda i: (i, 0)),
            pl.BlockSpec((be, LANES), lambda i: (i, 0)),
            pl.BlockSpec((be, DE), lambda i: (jnp.minimum(i, ea_last), 0)),
            full((DE, H)), full((1, H)), full((1, H)),
            full((H, H)), full((1, H)),
            full((H, H)), full((1, H)), full((H, 1)),
        ],
        out_specs=[
            pl.BlockSpec((be, H), lambda i: (i, 0)),
            pl.BlockSpec((be, 4), lambda i: (i, 0)),
        ],
        out_shape=[
            jax.ShapeDtypeStruct((Ep, H), jnp.float32),
            jax.ShapeDtypeStruct((Ep, 4), jnp.float32),
        ],
    )(ga, gb, geo, ea,
      we1e, we1r, be1, we2, be2, wc1, bc1, wc2)


def _scatter_sc(ef, t4t, row2d, col2d, z128, z1d, C=128):
    H = ef.shape[1]
    N = 10 * z128.shape[0]
    Ep = row2d.shape[0] * C
    ET = Ep // NS
    NCH = ET // C
    ZR = z128.shape[0]
    mesh = plsc.VectorSubcoreMesh(core_axis_name="c", subcore_axis_name="s")

    @functools.partial(
        pl.kernel,
        out_type=[
            jax.ShapeDtypeStruct((N, H), jnp.float32),
            jax.ShapeDtypeStruct((N, H), jnp.float32),
            jax.ShapeDtypeStruct((4, N), jnp.float32),
        ],
        mesh=mesh,
        compiler_params=_SC_PARAMS,
        scratch_types=[
            pltpu.VMEM((NCH, C), jnp.int32),
            pltpu.VMEM((C, H), jnp.float32),
            pltpu.VMEM((C, H), jnp.float32),
            pltpu.VMEM((4, C), jnp.float32),
            pltpu.VMEM((4, C), jnp.float32),
            pltpu.VMEM_SHARED((N, H), jnp.float32),
            pltpu.VMEM_SHARED((N,), jnp.float32),
            pltpu.VMEM_SHARED((N,), jnp.float32),
            pltpu.VMEM_SHARED((N,), jnp.float32),
            pltpu.VMEM_SHARED((N,), jnp.float32),
        ] + [pltpu.SemaphoreType.DMA] * 4,
    )
    def k(ef_hbm, t4t_hbm, row2d_hbm, col2d_hbm, z128_hbm, z1d_hbm,
          agg_hbm, oth_hbm, fcnt_hbm,
          idx2_v, fbuf0, fbuf1, tbuf0, tbuf1,
          shf, sh0, sh1, sh2, sh3,
          ls0, ls1, lt0, lt1):
        cid = lax.axis_index("c")
        sid = lax.axis_index("s")
        shcs = [sh0, sh1, sh2, sh3]
        zrows = pl.ds(sid * ZR, ZR)
        fbufs = [fbuf0, fbuf1]
        tbufs = [tbuf0, tbuf1]
        lss = [ls0, ls1]
        lts = [lt0, lt1]

        @pl.when(sid < 10)
        def _():
            pltpu.sync_copy(z128_hbm, shf.at[zrows])

        @pl.when(cid == 0)
        def _():
            for d in range(4):
                @pl.when(sid == d)
                def _():
                    pltpu.sync_copy(z1d_hbm, shcs[d])

        plsc.subcore_barrier()

        def run(idx_hbm, do_t):
            pltpu.sync_copy(idx_hbm.at[pl.ds(sid * NCH, NCH)], idx2_v)

            def l_cps(c, b):
                off = sid * ET + c * C
                cps = [pltpu.make_async_copy(
                    ef_hbm.at[pl.ds(off, C)], fbufs[b], lss[b])]
                if do_t:
                    cps.append(pltpu.make_async_copy(
                        t4t_hbm.at[pl.ds(0, 4), pl.ds(off, C)],
                        tbufs[b], lts[b]))
                return cps

            for cp in l_cps(0, 0):
                cp.start()

            def outer(s, _):
                for b in (0, 1):
                    c = 2 * s + b
                    for cp in l_cps(c, b):
                        cp.wait()

                    @pl.when(c + 1 < NCH)
                    def _():
                        for cp in l_cps(c + 1, 1 - b):
                            cp.start()

                    pltpu.sync_copy(fbufs[b], shf.at[idx2_v.at[c]],
                                    add=True)
                    if do_t:
                        for d in range(4):
                            pltpu.sync_copy(tbufs[b].at[d],
                                            shcs[d].at[idx2_v.at[c]],
                                            add=True)
                return 0

            lax.fori_loop(0, NCH // 2, outer, 0)

        @pl.when(cid == 0)
        def _():
            run(row2d_hbm, True)

        @pl.when(cid == 1)
        def _():
            run(col2d_hbm, False)

        plsc.subcore_barrier()

        @pl.when(sid < 10)
        def _():
            @pl.when(cid == 0)
            def _():
                pltpu.sync_copy(shf.at[zrows], agg_hbm.at[zrows])

            @pl.when(cid == 1)
            def _():
                pltpu.sync_copy(shf.at[zrows], oth_hbm.at[zrows])

        @pl.when(jnp.logical_and(cid == 0, sid < 4))
        def _():
            for d in range(4):
                @pl.when(sid == d)
                def _():
                    pltpu.sync_copy(shcs[d], fcnt_hbm.at[d])

    return k(ef, t4t, row2d, col2d, z128, z1d)


def _node_body(h_ref, x_ref, v_ref, agg_ref, oth_ref, fc_ref,
               wv1_ref, bv1_ref, wv2_ref, bv2_ref,
               wn1a_ref, wn1b_ref, wn1c_ref, bn1_ref, wn2_ref, bn2_ref,
               h_out, x_out, v_out):
    h = h_ref[...]
    fc = fc_ref[...]
    deg = jnp.maximum(fc[:, 3:4], 1.0)
    f = fc[:, 0:3] / deg
    sh = jnp.maximum(
        jnp.dot(h, wv1_ref[...], preferred_element_type=jnp.float32)
        + bv1_ref[...], 0.0)
    scale = jnp.dot(sh, wv2_ref[...],
                    preferred_element_type=jnp.float32) + bv2_ref[...]
    vn = scale * v_ref[...] + f
    v_out[...] = vn
    x_out[...] = x_ref[...] + vn
    nm = jnp.maximum(
        jnp.dot(oth_ref[...], wn1a_ref[...],
                preferred_element_type=jnp.float32)
        + jnp.dot(h, wn1b_ref[...], preferred_element_type=jnp.float32)
        + jnp.dot(agg_ref[...], wn1c_ref[...],
                  preferred_element_type=jnp.float32)
        + bn1_ref[...], 0.0)
    h_out[...] = h + jnp.dot(nm, wn2_ref[...],
                             preferred_element_type=jnp.float32) + bn2_ref[...]


def _node_tc(h, x, v, agg, oth, fcnt,
             wv1, bv1, wv2, bv2, wn1a, wn1b, wn1c, bn1, wn2, bn2, bn=400):
    N, D = h.shape
    H = wn2.shape[0]
    full = lambda shape: pl.BlockSpec(shape, lambda i: (0, 0))
    return pl.pallas_call(
        _node_body,
        grid=(N // bn,),
        in_specs=[
            pl.BlockSpec((bn, D), lambda i: (i, 0)),
            pl.BlockSpec((bn, 3), lambda i: (i, 0)),
            pl.BlockSpec((bn, 3), lambda i: (i, 0)),
            pl.BlockSpec((bn, H), lambda i: (i, 0)),
            pl.BlockSpec((bn, H), lambda i: (i, 0)),
            pl.BlockSpec((bn, 4), lambda i: (i, 0)),
            full((D, H)), full((1, H)), full((H, 1)), full((1, 1)),
            full((H, H)), full((D, H)), full((H, H)), full((1, H)),
            full((H, D)), full((1, D)),
        ],
        out_specs=[
            pl.BlockSpec((bn, D), lambda i: (i, 0)),
            pl.BlockSpec((bn, 3), lambda i: (i, 0)),
            pl.BlockSpec((bn, 3), lambda i: (i, 0)),
        ],
        out_shape=[
            jax.ShapeDtypeStruct((N, D), jnp.float32),
            jax.ShapeDtypeStruct((N, 3), jnp.float32),
            jax.ShapeDtypeStruct((N, 3), jnp.float32),
        ],
    )(h, x, v, agg, oth, fcnt,
      wv1, bv1, wv2, bv2, wn1a, wn1b, wn1c, bn1, wn2, bn2)


def kernel(h, x, v, edge_attr, We1, be1, We2, be2, Wc1, bc1, Wc2,
           Wv1, bv1, Wv2, bv2, Wn1, bn1, Wn2, bn2,
           edge_index, isolated_index):
    N, D = h.shape
    H = We2.shape[0]
    E = edge_index.shape[1]
    DE = edge_attr.shape[1]
    row, col = edge_index[0], edge_index[1]

    C = 128
    Ep = -(-E // (NW * C)) * (NW * C)
    pad = Ep - E
    rowp = jnp.concatenate([row, jnp.zeros((pad,), jnp.int32)])
    colp = jnp.concatenate([col, jnp.zeros((pad,), jnp.int32)])
    xa, xb, xc = x[:, 0], x[:, 1], x[:, 2]
    bf = jnp.bfloat16

    A, Bm = _pre_tc(h, We1[:D], We1[D:2 * D])
    ga, gb, geo = _gather_sc(A, Bm, xa, xb, xc, rowp, colp, C=C)
    ef, t4 = _edge_tc(E, ga, gb, geo, edge_attr,
                       We1[2 * D + 1:], We1[2 * D:2 * D + 1],
                       be1.reshape(1, H), We2.astype(bf),
                       be2.reshape(1, H),
                       Wc1.astype(bf), bc1.reshape(1, H), Wc2.astype(bf))
    z128 = jnp.zeros((N // 10, H), jnp.float32)
    z1d = jnp.zeros((N,), jnp.float32)
    agg, oth, fcnt = _scatter_sc(ef, t4.T, rowp.reshape(-1, C),
                                 colp.reshape(-1, C), z128, z1d, C=C)
    return _node_tc(h, x, v, agg, oth, fcnt.T,
                    Wv1, bv1.reshape(1, H), Wv2, bv2.reshape(1, 1),
                    Wn1[:H], Wn1[H:H + D], Wn1[H + D:],
                    bn1.reshape(1, H), Wn2, bn2.reshape(1, D))


# speedup vs baseline: 1.2166x; 1.1485x over previous
import functools

import jax
import jax.numpy as jnp
from jax import lax
from jax.experimental import pallas as pl
from jax.experimental.pallas import tpu as pltpu
from jax.experimental.pallas import tpu_sc as plsc

NC, NS, LANES = 2, 16, 16
NW = NC * NS
_SC_PARAMS = pltpu.CompilerParams(needs_layout_passes=False)


def _pre_body(h_ref, wa_ref, wb_ref, a_ref, b_ref):
    hb = h_ref[...]
    a_ref[...] = jnp.dot(hb, wa_ref[...], preferred_element_type=jnp.float32)
    b_ref[...] = jnp.dot(hb, wb_ref[...], preferred_element_type=jnp.float32)


def _pre_tc(h, wa, wb, bn=400):
    N, D = h.shape
    H = wa.shape[1]
    return pl.pallas_call(
        _pre_body,
        grid=(N // bn,),
        in_specs=[
            pl.BlockSpec((bn, D), lambda i: (i, 0)),
            pl.BlockSpec((D, H), lambda i: (0, 0)),
            pl.BlockSpec((D, H), lambda i: (0, 0)),
        ],
        out_specs=[
            pl.BlockSpec((bn, H), lambda i: (i, 0)),
            pl.BlockSpec((bn, H), lambda i: (i, 0)),
        ],
        out_shape=[
            jax.ShapeDtypeStruct((N, H), jnp.float32),
            jax.ShapeDtypeStruct((N, H), jnp.float32),
        ],
    )(h, wa, wb)


def _gather_sc(A, B, xa, xb, xc, rowp, colp, C=128, nch0=54):
    N, H = A.shape
    Ep = rowp.shape[0]
    NCHT = Ep // (NS * C)
    NCH0 = nch0
    NCH1 = NCHT - NCH0
    EW0, EW1 = NCH0 * C, NCH1 * C
    EWMAX = max(EW0, EW1)
    mesh = plsc.VectorSubcoreMesh(core_axis_name="c", subcore_axis_name="s")

    @functools.partial(
        pl.kernel,
        out_type=[
            jax.ShapeDtypeStruct((Ep, H), jnp.float32),
            jax.ShapeDtypeStruct((Ep, H), jnp.float32),
            jax.ShapeDtypeStruct((Ep, LANES), jnp.float32),
        ],
        mesh=mesh,
        compiler_params=_SC_PARAMS,
        scratch_types=[
            pltpu.VMEM((EWMAX,), jnp.int32),
            pltpu.VMEM((EWMAX,), jnp.int32),
            pltpu.VMEM((N,), jnp.float32),
            pltpu.VMEM((N,), jnp.float32),
            pltpu.VMEM((N,), jnp.float32),
            pltpu.VMEM((C, LANES), jnp.float32),
            pltpu.VMEM((C, H), jnp.float32),
            pltpu.VMEM((C, H), jnp.float32),
            pltpu.VMEM((C, H), jnp.float32),
            pltpu.VMEM((C, H), jnp.float32),
        ] + [pltpu.SemaphoreType.DMA] * 8,
    )
    def k(a_hbm, b_hbm, xa_hbm, xb_hbm, xc_hbm, row_hbm, col_hbm,
          ga_hbm, gb_hbm, geo_hbm,
          idxr_v, idxc_v, xa_v, xb_v, xc_v, geo_v,
          buf_a0, buf_a1, buf_b0, buf_b1,
          gsa0, gsa1, gsb0, gsb1, wsa0, wsa1, wsb0, wsb1):
        cid = lax.axis_index("c")
        sid = lax.axis_index("s")
        pltpu.sync_copy(xa_hbm, xa_v)
        pltpu.sync_copy(xb_hbm, xb_v)
        pltpu.sync_copy(xc_hbm, xc_v)
        lane = lax.iota(jnp.int32, LANES)
        xs = [xa_v, xb_v, xc_v]
        bufs_a = [buf_a0, buf_a1]
        bufs_b = [buf_b0, buf_b1]
        gsa = [gsa0, gsa1]
        gsb = [gsb0, gsb1]
        wsa = [wsa0, wsa1]
        wsb = [wsb0, wsb1]

        def pipeline(base, ew, nch):
            pltpu.sync_copy(row_hbm.at[pl.ds(base, ew)], idxr_v.at[pl.ds(0, ew)])
            pltpu.sync_copy(col_hbm.at[pl.ds(base, ew)], idxc_v.at[pl.ds(0, ew)])

            def geo_chunk(c):
                def grp(gi, _):
                    e0 = c * C + gi * LANES
                    ir = idxr_v[pl.ds(e0, LANES)]
                    ic = idxc_v[pl.ds(e0, LANES)]
                    g = gi * LANES + lane
                    rad = jnp.zeros((LANES,), jnp.float32)
                    for d in range(3):
                        dd = jnp.full((LANES,), d, jnp.int32)
                        diff = (plsc.load_gather(xs[d], [ir])
                                - plsc.load_gather(xs[d], [ic]))
                        plsc.store_scatter(geo_v, [g, dd], diff)
                        rad = rad + diff * diff
                    plsc.store_scatter(
                        geo_v, [g, jnp.full((LANES,), 3, jnp.int32)], rad)
                    return 0
                lax.fori_loop(0, C // LANES, grp, 0)
                pltpu.sync_copy(geo_v, geo_hbm.at[pl.ds(base + c * C, C)])

            def g_cp(c, b):
                off = c * C
                return (pltpu.make_async_copy(
                            a_hbm.at[idxr_v.at[pl.ds(off, C)]],
                            bufs_a[b], gsa[b]),
                        pltpu.make_async_copy(
                            b_hbm.at[idxc_v.at[pl.ds(off, C)]],
                            bufs_b[b], gsb[b]))

            def w_cp(c, b):
                off = base + c * C
                return (pltpu.make_async_copy(
                            bufs_a[b], ga_hbm.at[pl.ds(off, C)], wsa[b]),
                        pltpu.make_async_copy(
                            bufs_b[b], gb_hbm.at[pl.ds(off, C)], wsb[b]))

            for cp in g_cp(0, 0):
                cp.start()

            def outer(s, _):
                for b in (0, 1):
                    c = 2 * s + b
                    for cp in g_cp(c, b):
                        cp.wait()
                    for cp in w_cp(c, b):
                        cp.start()

                    @pl.when(c + 1 < nch)
                    def _():
                        @pl.when(c >= 1)
                        def _():
                            for cp in w_cp(c - 1, 1 - b):
                                cp.wait()
                        for cp in g_cp(c + 1, 1 - b):
                            cp.start()

                    geo_chunk(c)
                return 0

            lax.fori_loop(0, nch // 2, outer, 0)
            for cp in w_cp(nch - 2, 0):
                cp.wait()
            for cp in w_cp(nch - 1, 1):
                cp.wait()

        @pl.when(cid == 0)
        def _():
            pipeline(sid * (EW0 + EW1), EW0, NCH0)

        @pl.when(cid == 1)
        def _():
            pipeline(sid * (EW0 + EW1) + EW0, EW1, NCH1)

    return k(A, B, xa, xb, xc, rowp, colp)


def _edge_body(nedge, ga_ref, gb_ref, geo_ref, ea_ref,
               we1e_ref, we1r_ref, be1_ref, we2_ref, be2_ref,
               wc1_ref, bc1_ref, wc2_ref,
               ef_ref, t16_ref):
    be = ga_ref.shape[0]
    geo = geo_ref[...]
    radial = geo[:, 3:4]
    m = jnp.maximum(
        ga_ref[...] + gb_ref[...] + radial * we1r_ref[...]
        + jnp.dot(ea_ref[...], we1e_ref[...],
                  preferred_element_type=jnp.float32)
        + be1_ref[...], 0.0)
    ef = jnp.maximum(
        jnp.dot(m.astype(jnp.bfloat16), we2_ref[...],
                preferred_element_type=jnp.float32)
        + be2_ref[...], 0.0)
    ch = jnp.maximum(
        jnp.dot(ef.astype(jnp.bfloat16), wc1_ref[...],
                preferred_element_type=jnp.float32)
        + bc1_ref[...], 0.0)
    cm = jnp.dot(ch.astype(jnp.bfloat16), wc2_ref[...],
                 preferred_element_type=jnp.float32)
    t = jnp.clip(geo[:, 0:4] * cm, -100.0, 100.0)
    lane = lax.broadcasted_iota(jnp.int32, t.shape, 1)
    t4 = jnp.where(lane < 3, t, 1.0)
    eid = pl.program_id(0) * be + lax.broadcasted_iota(jnp.int32, (be, 1), 0)
    emask = eid < nedge
    ef_ref[...] = jnp.where(emask, ef, 0.0)
    t16_ref[...] = jnp.where(emask, t4, 0.0)


def _edge_tc(nedge, ga, gb, geo, ea,
             we1e, we1r, be1, we2, be2, wc1, bc1, wc2, be=1024):
    Ep, H = ga.shape
    DE = ea.shape[1]
    ea_last = (ea.shape[0] + be - 1) // be - 1
    full = lambda shape: pl.BlockSpec(shape, lambda i: (0, 0))
    return pl.pallas_call(
        functools.partial(_edge_body, nedge),
        grid=(Ep // be,),
        compiler_params=pltpu.CompilerParams(
            dimension_semantics=("parallel",)),
        in_specs=[
            pl.BlockSpec((be, H), lambda i: (i, 0)),
            pl.BlockSpec((be, H), lambda i: (i, 0)),
            pl.BlockSpec((be, LANES), lambda i: (i, 0)),
            pl.BlockSpec((be, DE), lambda i: (jnp.minimum(i, ea_last), 0)),
            full((DE, H)), full((1, H)), full((1, H)),
            full((H, H)), full((1, H)),
            full((H, H)), full((1, H)), full((H, 1)),
        ],
        out_specs=[
            pl.BlockSpec((be, H), lambda i: (i, 0)),
            pl.BlockSpec((be, 4), lambda i: (i, 0)),
        ],
        out_shape=[
            jax.ShapeDtypeStruct((Ep, H), jnp.float32),
            jax.ShapeDtypeStruct((Ep, 4), jnp.float32),
        ],
    )(ga, gb, geo, ea,
      we1e, we1r, be1, we2, be2, wc1, bc1, wc2)


def _scatter_sc(ef, t4t, rowp, colp, z128, z1d, C=128):
    H = ef.shape[1]
    N = 10 * z128.shape[0]
    Ep = rowp.shape[0]
    ET = Ep // NS
    NCH = ET // C
    ZR = z128.shape[0]
    mesh = plsc.VectorSubcoreMesh(core_axis_name="c", subcore_axis_name="s")

    @functools.partial(
        pl.kernel,
        out_type=[
            jax.ShapeDtypeStruct((N, H), jnp.float32),
            jax.ShapeDtypeStruct((N, H), jnp.float32),
            jax.ShapeDtypeStruct((4, N), jnp.float32),
        ],
        mesh=mesh,
        compiler_params=_SC_PARAMS,
        scratch_types=[
            pltpu.VMEM((2, C), jnp.int32),
            pltpu.VMEM((C, H), jnp.float32),
            pltpu.VMEM((C, H), jnp.float32),
            pltpu.VMEM((4, C), jnp.float32),
            pltpu.VMEM((4, C), jnp.float32),
            pltpu.VMEM_SHARED((N, H), jnp.float32),
            pltpu.VMEM_SHARED((N,), jnp.float32),
            pltpu.VMEM_SHARED((N,), jnp.float32),
            pltpu.VMEM_SHARED((N,), jnp.float32),
            pltpu.VMEM_SHARED((N,), jnp.float32),
        ] + [pltpu.SemaphoreType.DMA] * 6,
    )
    def k(ef_hbm, t4t_hbm, row_hbm, col_hbm, z128_hbm, z1d_hbm,
          agg_hbm, oth_hbm, fcnt_hbm,
          idx2_v, fbuf0, fbuf1, tbuf0, tbuf1,
          shf, sh0, sh1, sh2, sh3,
          ls0, ls1, lt0, lt1, li0, li1):
        cid = lax.axis_index("c")
        sid = lax.axis_index("s")
        shcs = [sh0, sh1, sh2, sh3]
        zrows = pl.ds(sid * ZR, ZR)
        fbufs = [fbuf0, fbuf1]
        tbufs = [tbuf0, tbuf1]
        lss = [ls0, ls1]
        lts = [lt0, lt1]
        lis = [li0, li1]

        @pl.when(sid < 10)
        def _():
            pltpu.sync_copy(z128_hbm, shf.at[zrows])

        @pl.when(cid == 0)
        def _():
            for d in range(4):
                @pl.when(sid == d)
                def _():
                    pltpu.sync_copy(z1d_hbm, shcs[d])

        plsc.subcore_barrier()

        def run(idx_hbm, do_t):
            def l_cps(c, b):
                off = sid * ET + c * C
                cps = [pltpu.make_async_copy(
                           ef_hbm.at[pl.ds(off, C)], fbufs[b], lss[b]),
                       pltpu.make_async_copy(
                           idx_hbm.at[pl.ds(off, C)], idx2_v.at[b], lis[b])]
                if do_t:
                    cps.append(pltpu.make_async_copy(
                        t4t_hbm.at[pl.ds(0, 4), pl.ds(off, C)],
                        tbufs[b], lts[b]))
                return cps

            for cp in l_cps(0, 0):
                cp.start()

            def outer(s, _):
                for b in (0, 1):
                    c = 2 * s + b
                    for cp in l_cps(c, b):
                        cp.wait()

                    @pl.when(c + 1 < NCH)
                    def _():
                        for cp in l_cps(c + 1, 1 - b):
                            cp.start()

                    pltpu.sync_copy(fbufs[b], shf.at[idx2_v.at[b]],
                                    add=True)
                    if do_t:
                        for d in range(4):
                            pltpu.sync_copy(tbufs[b].at[d],
                                            shcs[d].at[idx2_v.at[b]],
                                            add=True)
                return 0

            lax.fori_loop(0, NCH // 2, outer, 0)

        @pl.when(cid == 0)
        def _():
            run(row_hbm, True)

        @pl.when(cid == 1)
        def _():
            run(col_hbm, False)

        plsc.subcore_barrier()

        @pl.when(sid < 10)
        def _():
            @pl.when(cid == 0)
            def _():
                pltpu.sync_copy(shf.at[zrows], agg_hbm.at[zrows])

            @pl.when(cid == 1)
            def _():
                pltpu.sync_copy(shf.at[zrows], oth_hbm.at[zrows])

        @pl.when(jnp.logical_and(cid == 0, sid < 4))
        def _():
            for d in range(4):
                @pl.when(sid == d)
                def _():
                    pltpu.sync_copy(shcs[d], fcnt_hbm.at[d])

    return k(ef, t4t, rowp, colp, z128, z1d)


def _node_body(h_ref, x_ref, v_ref, agg_ref, oth_ref, fc_ref,
               wv1_ref, bv1_ref, wv2_ref, bv2_ref,
               wn1a_ref, wn1b_ref, wn1c_ref, bn1_ref, wn2_ref, bn2_ref,
               h_out, x_out, v_out):
    h = h_ref[...]
    fc = fc_ref[...]
    deg = jnp.maximum(fc[:, 3:4], 1.0)
    f = fc[:, 0:3] / deg
    sh = jnp.maximum(
        jnp.dot(h, wv1_ref[...], preferred_element_type=jnp.float32)
        + bv1_ref[...], 0.0)
    scale = jnp.dot(sh, wv2_ref[...],
                    preferred_element_type=jnp.float32) + bv2_ref[...]
    vn = scale * v_ref[...] + f
    v_out[...] = vn
    x_out[...] = x_ref[...] + vn
    nm = jnp.maximum(
        jnp.dot(oth_ref[...], wn1a_ref[...],
                preferred_element_type=jnp.float32)
        + jnp.dot(h, wn1b_ref[...], preferred_element_type=jnp.float32)
        + jnp.dot(agg_ref[...], wn1c_ref[...],
                  preferred_element_type=jnp.float32)
        + bn1_ref[...], 0.0)
    h_out[...] = h + jnp.dot(nm, wn2_ref[...],
                             preferred_element_type=jnp.float32) + bn2_ref[...]


def _node_tc(h, x, v, agg, oth, fcnt,
             wv1, bv1, wv2, bv2, wn1a, wn1b, wn1c, bn1, wn2, bn2, bn=400):
    N, D = h.shape
    H = wn2.shape[0]
    full = lambda shape: pl.BlockSpec(shape, lambda i: (0, 0))
    return pl.pallas_call(
        _node_body,
        grid=(N // bn,),
        in_specs=[
            pl.BlockSpec((bn, D), lambda i: (i, 0)),
            pl.BlockSpec((bn, 3), lambda i: (i, 0)),
            pl.BlockSpec((bn, 3), lambda i: (i, 0)),
            pl.BlockSpec((bn, H), lambda i: (i, 0)),
            pl.BlockSpec((bn, H), lambda i: (i, 0)),
            pl.BlockSpec((bn, 4), lambda i: (i, 0)),
            full((D, H)), full((1, H)), full((H, 1)), full((1, 1)),
            full((H, H)), full((D, H)), full((H, H)), full((1, H)),
            full((H, D)), full((1, D)),
        ],
        out_specs=[
            pl.BlockSpec((bn, D), lambda i: (i, 0)),
            pl.BlockSpec((bn, 3), lambda i: (i, 0)),
            pl.BlockSpec((bn, 3), lambda i: (i, 0)),
        ],
        out_shape=[
            jax.ShapeDtypeStruct((N, D), jnp.float32),
            jax.ShapeDtypeStruct((N, 3), jnp.float32),
            jax.ShapeDtypeStruct((N, 3), jnp.float32),
        ],
    )(h, x, v, agg, oth, fcnt,
      wv1, bv1, wv2, bv2, wn1a, wn1b, wn1c, bn1, wn2, bn2)


def kernel(h, x, v, edge_attr, We1, be1, We2, be2, Wc1, bc1, Wc2,
           Wv1, bv1, Wv2, bv2, Wn1, bn1, Wn2, bn2,
           edge_index, isolated_index):
    N, D = h.shape
    H = We2.shape[0]
    E = edge_index.shape[1]
    DE = edge_attr.shape[1]
    row, col = edge_index[0], edge_index[1]

    C = 128
    Ep = -(-E // (NW * C)) * (NW * C)
    pad = Ep - E
    rowp = jnp.concatenate([row, jnp.zeros((pad,), jnp.int32)])
    colp = jnp.concatenate([col, jnp.zeros((pad,), jnp.int32)])
    xa, xb, xc = x[:, 0], x[:, 1], x[:, 2]
    bf = jnp.bfloat16

    A, Bm = _pre_tc(h, We1[:D], We1[D:2 * D])
    ga, gb, geo = _gather_sc(A, Bm, xa, xb, xc, rowp, colp, C=C)
    ef, t4 = _edge_tc(E, ga, gb, geo, edge_attr,
                       We1[2 * D + 1:], We1[2 * D:2 * D + 1],
                       be1.reshape(1, H), We2.astype(bf),
                       be2.reshape(1, H),
                       Wc1.astype(bf), bc1.reshape(1, H), Wc2.astype(bf))
    z128 = jnp.zeros((N // 10, H), jnp.float32)
    z1d = jnp.zeros((N,), jnp.float32)
    agg, oth, fcnt = _scatter_sc(ef, t4.T, rowp, colp, z128, z1d, C=C)
    return _node_tc(h, x, v, agg, oth, fcnt.T,
                    Wv1, bv1.reshape(1, H), Wv2, bv2.reshape(1, 1),
                    Wn1[:H], Wn1[H:H + D], Wn1[H + D:],
                    bn1.reshape(1, H), Wn2, bn2.reshape(1, D))
